# Initial kernel scaffold; baseline (speedup 1.0000x reference)
import functools

import jax
import jax.numpy as jnp
from jax import lax
from jax.experimental import pallas as pl
from jax.experimental.pallas import tpu as pltpu
from jax.experimental.pallas import tpu_sc as plsc

INPUT_DIM = 2048
OUTPUT_DIM = 512
K_DIM = 256
N_KEYS = 512
HEADS = 4
KNN = 8
HALF = K_DIM // 2
NSLOT = HEADS * KNN

NEG = jnp.float32(-1e30)


BB = 512


def _extract_top8(s, vals, idxs, idx_src=None):
    n = s.shape[1]
    iota = lax.broadcasted_iota(jnp.int32, s.shape, 1)
    for _ in range(KNN):
        m = jnp.max(s, axis=1, keepdims=True)
        p = jnp.min(jnp.where(s == m, iota, n), axis=1, keepdims=True)
        hit = iota == p
        if idx_src is None:
            idxs.append(p)
        else:
            idxs.append(jnp.sum(jnp.where(hit, idx_src, 0), axis=1, keepdims=True))
        vals.append(m)
        s = jnp.where(hit, NEG, s)
    return s


def _tc_body(x_ref, wq_ref, bq_ref, keys_ref, idx_ref, wb_ref):
    q = lax.dot_general(
        x_ref[...], wq_ref[...], (((1,), (1,)), ((), ())),
        preferred_element_type=jnp.float32,
    ) + bq_ref[...]

    idx_cols = []
    wb_cols = []
    for h in range(HEADS):
        q1 = q[:, h * K_DIM: h * K_DIM + HALF]
        q2 = q[:, h * K_DIM + HALF: (h + 1) * K_DIM]
        s1 = lax.dot_general(
            q1, keys_ref[h, 0], (((1,), (1,)), ((), ())),
            preferred_element_type=jnp.float32)
        s2 = lax.dot_general(
            q2, keys_ref[h, 1], (((1,), (1,)), ((), ())),
            preferred_element_type=jnp.float32)

        v1, i1 = [], []
        _extract_top8(s1, v1, i1)
        v2, i2 = [], []
        _extract_top8(s2, v2, i2)

        s2cat = jnp.concatenate(v2, axis=1)
        i2cat = jnp.concatenate(i2, axis=1)
        all_s = jnp.concatenate([v1[i] + s2cat for i in range(KNN)], axis=1)
        all_i = jnp.concatenate(
            [i1[i] * N_KEYS + i2cat for i in range(KNN)], axis=1)

        sv, si = [], []
        _extract_top8(all_s, sv, si, idx_src=all_i)
        svc = jnp.concatenate(sv, axis=1)
        e = jnp.exp(svc - svc[:, 0:1])
        w = e / jnp.sum(e, axis=1, keepdims=True)

        idx_cols.extend(si)
        for k in range(KNN):
            wb_cols.append(jnp.broadcast_to(w[:, k:k + 1], (BB, 16)))

    idx_ref[...] = jnp.concatenate(idx_cols, axis=1)
    wb_ref[...] = jnp.concatenate(wb_cols, axis=1)


def _tc_call(x, Wq, bq2, keys):
    b = x.shape[0]
    return pl.pallas_call(
        _tc_body,
        grid=(b // BB,),
        in_specs=[
            pl.BlockSpec((BB, INPUT_DIM), lambda i: (i, 0)),
            pl.BlockSpec((HEADS * K_DIM, INPUT_DIM), lambda i: (0, 0)),
            pl.BlockSpec((1, HEADS * K_DIM), lambda i: (0, 0)),
            pl.BlockSpec((HEADS, 2, N_KEYS, HALF), lambda i: (0, 0, 0, 0)),
        ],
        out_specs=[
            pl.BlockSpec((BB, NSLOT), lambda i: (i, 0)),
            pl.BlockSpec((BB, 16 * NSLOT), lambda i: (i, 0)),
        ],
        out_shape=[
            jax.ShapeDtypeStruct((b, NSLOT), jnp.int32),
            jax.ShapeDtypeStruct((b, 16 * NSLOT), jnp.float32),
        ],
    )(x, Wq, bq2, keys)


NC, NS, L = 2, 16, 16
NW = NC * NS
C = 2
DB = OUTPUT_DIM // L


def _sc_embed(values, idx2, wb, b):
    bpw = b // NW
    nchunk = bpw // C

    mesh = plsc.VectorSubcoreMesh(core_axis_name="c", subcore_axis_name="s")

    @functools.partial(
        pl.kernel,
        out_type=jax.ShapeDtypeStruct((b, OUTPUT_DIM), jnp.float32),
        mesh=mesh,
        scratch_types=[
            pltpu.VMEM((nchunk, C * NSLOT), jnp.int32),
            pltpu.VMEM((2, C * NSLOT, OUTPUT_DIM), jnp.float32),
            pltpu.VMEM((2, C, OUTPUT_DIM), jnp.float32),
            pltpu.VMEM((2, C, OUTPUT_DIM), jnp.float32),
            pltpu.SemaphoreType.DMA((2,)),
            pltpu.SemaphoreType.DMA((2,)),
            pltpu.SemaphoreType.DMA((2,)),
        ],
    )
    def k(values_hbm, idx_hbm, wb_hbm, out_hbm,
          idx_v, rows_v, wbuf, obuf, sem_g, sem_w, sem_o):
        wid = lax.axis_index("s") * NC + lax.axis_index("c")
        base_chunk = wid * nchunk
        base_item = wid * bpw

        pltpu.sync_copy(idx_hbm.at[pl.ds(base_chunk, nchunk)], idx_v)

        def start(g, buf):
            pltpu.async_copy(values_hbm.at[idx_v.at[g]], rows_v.at[buf],
                             sem_g.at[buf])
            pltpu.async_copy(wb_hbm.at[pl.ds(base_item + g * C, C)],
                             wbuf.at[buf], sem_w.at[buf])

        def compute(g, buf):
            for ci in range(C):
                def kbody(kk, accs):
                    wv = wbuf[buf, ci, pl.ds(kk * L, L)]
                    row = ci * NSLOT + kk
                    return tuple(
                        accs[d] + rows_v[buf, row, pl.ds(d * L, L)] * wv
                        for d in range(DB))
                accs = lax.fori_loop(
                    0, NSLOT, kbody,
                    tuple(jnp.zeros((L,), jnp.float32) for _ in range(DB)))
                for d in range(DB):
                    obuf[buf, ci, pl.ds(d * L, L)] = accs[d]

        start(0, 0)
        start(1, 1)

        def outer(g2, carry):
            for buf in range(2):
                g = g2 * 2 + buf
                item0 = base_item + g * C
                pltpu.make_async_copy(values_hbm.at[idx_v.at[g]],
                                      rows_v.at[buf], sem_g.at[buf]).wait()
                pltpu.make_async_copy(wb_hbm.at[pl.ds(item0, C)],
                                      wbuf.at[buf], sem_w.at[buf]).wait()

                @pl.when(g2 > 0)
                def _():
                    pltpu.make_async_copy(
                        obuf.at[buf], out_hbm.at[pl.ds(item0 - 2 * C, C)],
                        sem_o.at[buf]).wait()

                compute(g, buf)

                @pl.when(g2 < nchunk // 2 - 1)
                def _():
                    start(g + 2, buf)

                pltpu.async_copy(obuf.at[buf], out_hbm.at[pl.ds(item0, C)],
                                 sem_o.at[buf])
            return carry

        lax.fori_loop(0, nchunk // 2, outer, 0)

        for buf in range(2):
            g = nchunk - 2 + buf
            pltpu.make_async_copy(
                obuf.at[buf], out_hbm.at[pl.ds(base_item + g * C, C)],
                sem_o.at[buf]).wait()

    return k(values, idx2, wb)


def kernel(x, Wq, bq, keys, values):
    b = x.shape[0]
    idx, wb = _tc_call(x, Wq, bq.reshape(1, -1), keys)
    idx2 = idx.reshape(b // C, C * NSLOT)
    return _sc_embed(values, idx2, wb, b)


# speedup vs baseline: 7.7996x; 7.7996x over previous
import functools

import jax
import jax.numpy as jnp
from jax import lax
from jax.experimental import pallas as pl
from jax.experimental.pallas import tpu as pltpu
from jax.experimental.pallas import tpu_sc as plsc

INPUT_DIM = 2048
OUTPUT_DIM = 512
K_DIM = 256
N_KEYS = 512
HEADS = 4
KNN = 8
HALF = K_DIM // 2
NSLOT = HEADS * KNN

NEG = -1e30


BB = 512


def _extract_top8(s, vals, idxs, idx_src=None):
    n = s.shape[1]
    iota = lax.broadcasted_iota(jnp.int32, s.shape, 1)
    for _ in range(KNN):
        m = jnp.max(s, axis=1, keepdims=True)
        p = jnp.min(jnp.where(s == m, iota, n), axis=1, keepdims=True)
        hit = iota == p
        if idx_src is None:
            idxs.append(p)
        else:
            idxs.append(jnp.sum(jnp.where(hit, idx_src, 0), axis=1, keepdims=True))
        vals.append(m)
        s = jnp.where(hit, NEG, s)
    return s


def _tc_body(x_ref, wq_ref, bq_ref, keys_ref, idx_ref, wb_ref):
    q = lax.dot_general(
        x_ref[...], wq_ref[...], (((1,), (1,)), ((), ())),
        preferred_element_type=jnp.float32,
    ) + bq_ref[...]

    idx_cols = []
    wb_cols = []
    for h in range(HEADS):
        q1 = q[:, h * K_DIM: h * K_DIM + HALF]
        q2 = q[:, h * K_DIM + HALF: (h + 1) * K_DIM]
        s1 = lax.dot_general(
            q1, keys_ref[h, 0], (((1,), (1,)), ((), ())),
            preferred_element_type=jnp.float32)
        s2 = lax.dot_general(
            q2, keys_ref[h, 1], (((1,), (1,)), ((), ())),
            preferred_element_type=jnp.float32)

        v1, i1 = [], []
        _extract_top8(s1, v1, i1)
        v2, i2 = [], []
        _extract_top8(s2, v2, i2)

        s2cat = jnp.concatenate(v2, axis=1)
        i2cat = jnp.concatenate(i2, axis=1)
        all_s = jnp.concatenate([v1[i] + s2cat for i in range(KNN)], axis=1)
        all_i = jnp.concatenate(
            [i1[i] * N_KEYS + i2cat for i in range(KNN)], axis=1)

        sv, si = [], []
        _extract_top8(all_s, sv, si, idx_src=all_i)
        svc = jnp.concatenate(sv, axis=1)
        e = jnp.exp(svc - svc[:, 0:1])
        w = e / jnp.sum(e, axis=1, keepdims=True)

        idx_cols.extend(si)
        for k in range(KNN):
            wb_cols.append(jnp.broadcast_to(w[:, k:k + 1], (BB, 16)))

    idx_ref[...] = jnp.concatenate(idx_cols, axis=1)
    wb_ref[...] = jnp.concatenate(wb_cols, axis=1)


def _tc_call(x, Wq, bq2, keys):
    b = x.shape[0]
    return pl.pallas_call(
        _tc_body,
        grid=(b // BB,),
        in_specs=[
            pl.BlockSpec((BB, INPUT_DIM), lambda i: (i, 0)),
            pl.BlockSpec((HEADS * K_DIM, INPUT_DIM), lambda i: (0, 0)),
            pl.BlockSpec((1, HEADS * K_DIM), lambda i: (0, 0)),
            pl.BlockSpec((HEADS, 2, N_KEYS, HALF), lambda i: (0, 0, 0, 0)),
        ],
        out_specs=[
            pl.BlockSpec((BB, NSLOT), lambda i: (i, 0)),
            pl.BlockSpec((BB, 16 * NSLOT), lambda i: (i, 0)),
        ],
        out_shape=[
            jax.ShapeDtypeStruct((b, NSLOT), jnp.int32),
            jax.ShapeDtypeStruct((b, 16 * NSLOT), jnp.float32),
        ],
    )(x, Wq, bq2, keys)


NC, NS, L = 2, 16, 16
NW = NC * NS
C = 2
DB = OUTPUT_DIM // L


def _sc_embed(values, idx2, wb, b):
    bpw = b // NW
    nchunk = bpw // C

    mesh = plsc.VectorSubcoreMesh(
        core_axis_name="c", subcore_axis_name="s",
        num_cores=NC, num_subcores=NS)

    @functools.partial(
        pl.kernel,
        out_type=jax.ShapeDtypeStruct((b, OUTPUT_DIM), jnp.float32),
        mesh=mesh,
        scratch_types=[
            pltpu.VMEM((nchunk, C * NSLOT), jnp.int32),
            pltpu.VMEM((2, C * NSLOT, OUTPUT_DIM), jnp.float32),
            pltpu.VMEM((2, C, OUTPUT_DIM), jnp.float32),
            pltpu.VMEM((2, C, OUTPUT_DIM), jnp.float32),
            pltpu.SemaphoreType.DMA((2,)),
            pltpu.SemaphoreType.DMA((2,)),
            pltpu.SemaphoreType.DMA((2,)),
        ],
    )
    def k(values_hbm, idx_hbm, wb_hbm, out_hbm,
          idx_v, rows_v, wbuf, obuf, sem_g, sem_w, sem_o):
        wid = lax.axis_index("s") * NC + lax.axis_index("c")
        base_chunk = wid * nchunk
        base_item = wid * bpw

        pltpu.sync_copy(idx_hbm.at[pl.ds(base_chunk, nchunk)], idx_v)

        def start(g, buf):
            pltpu.async_copy(values_hbm.at[idx_v.at[g]], rows_v.at[buf],
                             sem_g.at[buf])
            pltpu.async_copy(wb_hbm.at[pl.ds(base_item + g * C, C)],
                             wbuf.at[buf], sem_w.at[buf])

        def compute(g, buf):
            for ci in range(C):
                def kbody(kk, accs):
                    wv = wbuf[buf, ci, pl.ds(kk * L, L)]
                    row = ci * NSLOT + kk
                    return tuple(
                        accs[d] + rows_v[buf, row, pl.ds(d * L, L)] * wv
                        for d in range(DB))
                accs = lax.fori_loop(
                    0, NSLOT, kbody,
                    tuple(jnp.zeros((L,), jnp.float32) for _ in range(DB)))
                for d in range(DB):
                    obuf[buf, ci, pl.ds(d * L, L)] = accs[d]

        start(0, 0)
        start(1, 1)

        def outer(g2, carry):
            for buf in range(2):
                g = g2 * 2 + buf
                item0 = base_item + g * C
                pltpu.make_async_copy(values_hbm.at[idx_v.at[g]],
                                      rows_v.at[buf], sem_g.at[buf]).wait()
                pltpu.make_async_copy(wb_hbm.at[pl.ds(item0, C)],
                                      wbuf.at[buf], sem_w.at[buf]).wait()

                @pl.when(g2 > 0)
                def _():
                    pltpu.make_async_copy(
                        obuf.at[buf], out_hbm.at[pl.ds(item0 - 2 * C, C)],
                        sem_o.at[buf]).wait()

                compute(g, buf)

                @pl.when(g2 < nchunk // 2 - 1)
                def _():
                    start(g + 2, buf)

                pltpu.async_copy(obuf.at[buf], out_hbm.at[pl.ds(item0, C)],
                                 sem_o.at[buf])
            return carry

        lax.fori_loop(0, nchunk // 2, outer, 0)

        for buf in range(2):
            g = nchunk - 2 + buf
            pltpu.make_async_copy(
                obuf.at[buf], out_hbm.at[pl.ds(base_item + g * C, C)],
                sem_o.at[buf]).wait()

    return k(values, idx2, wb)


def kernel(x, Wq, bq, keys, values):
    b = x.shape[0]
    idx, wb = _tc_call(x, Wq, bq.reshape(1, -1), keys)
    idx2 = idx.reshape(b // C, C * NSLOT)
    return _sc_embed(values, idx2, wb, b)
